# baseline (device time: 7280 ns/iter reference)
import jax
import jax.numpy as jnp
from jax import lax
from jax.experimental import pallas as pl
from jax.experimental.pallas import tpu as pltpu

NX, NY, NZ = 2, 4, 4


def kernel(u):
    sx, sy, sz = u.shape
    dtype = u.dtype

    bf16 = jnp.bfloat16

    def body(u_ref, out_ref, xsend, hx, hy, hz, ysend, zsend, send_sems,
             recv_sems, ready_sems):
        ix = lax.axis_index("x")
        iy = lax.axis_index("y")
        iz = lax.axis_index("z")

        dirs = [
            (ix > 0, (ix - 1, iy, iz),
             xsend.at[0], hx.at[1], hx.at[0]),
            (ix < NX - 1, (ix + 1, iy, iz),
             xsend.at[1], hx.at[0], hx.at[1]),
            (iy > 0, (ix, iy - 1, iz),
             ysend.at[0], hy.at[1], hy.at[0]),
            (iy < NY - 1, (ix, iy + 1, iz),
             ysend.at[1], hy.at[0], hy.at[1]),
            (iz > 0, (ix, iy, iz - 1),
             zsend.at[0], hz.at[1], hz.at[0]),
            (iz < NZ - 1, (ix, iy, iz + 1),
             zsend.at[1], hz.at[0], hz.at[1]),
        ]

        barrier = pltpu.get_barrier_semaphore()
        pl.semaphore_signal(barrier, inc=1)
        pl.semaphore_wait(barrier, 1)

        for d, (cond, nbr, _, _, _) in enumerate(dirs):
            @pl.when(cond)
            def _(d=d, nbr=nbr):
                pl.semaphore_signal(
                    ready_sems.at[d ^ 1], inc=1,
                    device_id=nbr, device_id_type=pl.DeviceIdType.MESH,
                )

        v = u_ref[:, :, :].astype(bf16)
        xsend[0:1] = v[0:1, :, :]
        xsend[1:2] = v[sx - 1:sx, :, :]
        ysend[0:1] = v[:, 0:1, :].reshape(1, sx, sz)
        ysend[1:2] = v[:, sy - 1:sy, :].reshape(1, sx, sz)
        zsend[0:1] = v[:, :, 0:1].reshape(1, sx, sy)
        zsend[1:2] = v[:, :, sz - 1:sz].reshape(1, sx, sy)

        for d, (cond, nbr, src, dst, _) in enumerate(dirs):
            @pl.when(cond)
            def _(d=d, nbr=nbr, src=src, dst=dst):
                pl.semaphore_wait(ready_sems.at[d], 1)
                pltpu.make_async_remote_copy(
                    src_ref=src, dst_ref=dst,
                    send_sem=send_sems.at[d], recv_sem=recv_sems.at[d ^ 1],
                    device_id=nbr, device_id_type=pl.DeviceIdType.MESH,
                ).start()

        zx = jnp.zeros((1, sy, sz), bf16)
        zy = jnp.zeros((sx, 1, sz), bf16)
        zz = jnp.zeros((sx, sy, 1), bf16)
        out_ref[:, :, :] = (
            jnp.concatenate([zx, v[:-1]], axis=0)
            + jnp.concatenate([v[1:], zx], axis=0)
            + jnp.concatenate([zy, v[:, :-1]], axis=1)
            + jnp.concatenate([v[:, 1:], zy], axis=1)
            + jnp.concatenate([zz, v[:, :, :-1]], axis=2)
            + jnp.concatenate([v[:, :, 1:], zz], axis=2)
            - 6.0 * v
        ).astype(dtype)

        for d, (cond, nbr, src, dst, myhalo) in enumerate(dirs):
            @pl.when(cond)
            def _(d=d, nbr=nbr, src=src, dst=dst, myhalo=myhalo):
                pltpu.make_async_remote_copy(
                    src_ref=src, dst_ref=myhalo,
                    send_sem=send_sems.at[d], recv_sem=recv_sems.at[d],
                    device_id=nbr, device_id_type=pl.DeviceIdType.MESH,
                ).wait_recv()
                pltpu.make_async_remote_copy(
                    src_ref=src, dst_ref=dst,
                    send_sem=send_sems.at[d], recv_sem=recv_sems.at[d ^ 1],
                    device_id=nbr, device_id_type=pl.DeviceIdType.MESH,
                ).wait_send()

        @pl.when(ix > 0)
        def _():
            out_ref[0:1, :, :] = (
                out_ref[0:1, :, :] + hx[0].reshape(1, sy, sz).astype(dtype)
            )

        @pl.when(ix < NX - 1)
        def _():
            out_ref[sx - 1:sx, :, :] = (
                out_ref[sx - 1:sx, :, :] + hx[1].reshape(1, sy, sz).astype(dtype)
            )

        @pl.when(iy > 0)
        def _():
            out_ref[:, 0:1, :] = (
                out_ref[:, 0:1, :] + hy[0].reshape(sx, 1, sz).astype(dtype)
            )

        @pl.when(iy < NY - 1)
        def _():
            out_ref[:, sy - 1:sy, :] = (
                out_ref[:, sy - 1:sy, :] + hy[1].reshape(sx, 1, sz).astype(dtype)
            )

        @pl.when(iz > 0)
        def _():
            out_ref[:, :, 0:1] = (
                out_ref[:, :, 0:1] + hz[0].reshape(sx, sy, 1).astype(dtype)
            )

        @pl.when(iz < NZ - 1)
        def _():
            out_ref[:, :, sz - 1:sz] = (
                out_ref[:, :, sz - 1:sz] + hz[1].reshape(sx, sy, 1).astype(dtype)
            )

        @pl.when(ix == 0)
        def _():
            out_ref[0:1, :, :] = jnp.zeros((1, sy, sz), dtype)

        @pl.when(ix == NX - 1)
        def _():
            out_ref[sx - 1:sx, :, :] = jnp.zeros((1, sy, sz), dtype)

        @pl.when(iy == 0)
        def _():
            out_ref[:, 0:1, :] = jnp.zeros((sx, 1, sz), dtype)

        @pl.when(iy == NY - 1)
        def _():
            out_ref[:, sy - 1:sy, :] = jnp.zeros((sx, 1, sz), dtype)

        @pl.when(iz == 0)
        def _():
            out_ref[:, :, 0:1] = jnp.zeros((sx, sy, 1), dtype)

        @pl.when(iz == NZ - 1)
        def _():
            out_ref[:, :, sz - 1:sz] = jnp.zeros((sx, sy, 1), dtype)

    return pl.pallas_call(
        body,
        out_shape=jax.ShapeDtypeStruct((sx, sy, sz), dtype),
        in_specs=[pl.BlockSpec(memory_space=pltpu.VMEM)],
        out_specs=pl.BlockSpec(memory_space=pltpu.VMEM),
        scratch_shapes=[
            pltpu.VMEM((2, sy, sz), jnp.bfloat16),
            pltpu.VMEM((2, sy, sz), jnp.bfloat16),
            pltpu.VMEM((2, sx, sz), jnp.bfloat16),
            pltpu.VMEM((2, sx, sy), jnp.bfloat16),
            pltpu.VMEM((2, sx, sz), jnp.bfloat16),
            pltpu.VMEM((2, sx, sy), jnp.bfloat16),
            pltpu.SemaphoreType.DMA((6,)),
            pltpu.SemaphoreType.DMA((6,)),
            pltpu.SemaphoreType.REGULAR((6,)),
        ],
        compiler_params=pltpu.CompilerParams(collective_id=0),
    )(u)
